# Initial kernel scaffold; baseline (speedup 1.0000x reference)
import jax
import jax.numpy as jnp
from jax.experimental import pallas as pl


def kernel(x, edge_index, edge_attr, W, b, gamma, beta):
    raise NotImplementedError("write your pallas kernel here")


# speedup vs baseline: 11.6794x; 11.6794x over previous
import jax
import jax.numpy as jnp
from jax import lax
from jax.experimental import pallas as pl
from jax.experimental.pallas import tpu as pltpu
from jax.experimental.pallas import tpu_sc as plsc

N_NODES = 10000
N_EDGES = 320000
D = 128

NC = 2
NS = 16
NW = NC * NS
LANES = 128
CH = 80
E_PAD = NW * CH * LANES
N_TRASH = N_NODES
N_SH = 10240
RPS = N_SH // NS

_MESH = plsc.VectorSubcoreMesh(
    core_axis_name="c", subcore_axis_name="s", num_cores=NC, num_subcores=NS
)


def _deg_body(col_hbm, zero128_hbm, deg_hbm, cidx_v, cidx1, ones_v, hist_sh):
    cid = lax.axis_index("c")
    sid = lax.axis_index("s")
    wid = cid * NS + sid

    pltpu.sync_copy(zero128_hbm, hist_sh.at[pl.ds(sid * RPS, RPS)])

    def fill_ones(i, _):
        for k in range(D // 16):
            ones_v[i, pl.ds(k * 16, 16)] = jnp.ones((16,), jnp.float32)
        return 0

    lax.fori_loop(0, LANES, fill_ones, 0)
    plsc.subcore_barrier()

    pltpu.sync_copy(col_hbm.at[wid], cidx_v)

    def chunk(j, _):
        for i in range(LANES // 16):
            cidx1[pl.ds(i * 16, 16)] = cidx_v[j, pl.ds(i * 16, 16)]
        pltpu.sync_copy(ones_v, hist_sh.at[cidx1], add=True)
        return 0

    lax.fori_loop(0, CH, chunk, 0)

    plsc.subcore_barrier()
    pltpu.sync_copy(
        hist_sh.at[pl.ds(sid * RPS, RPS)],
        deg_hbm.at[cid, pl.ds(sid * RPS, RPS)],
    )


_deg_call = pl.kernel(
    _deg_body,
    out_type=jax.ShapeDtypeStruct((NC, N_SH, D), jnp.float32),
    mesh=_MESH,
    scratch_types=[
        pltpu.VMEM((CH, LANES), jnp.int32),
        pltpu.VMEM((LANES,), jnp.int32),
        pltpu.VMEM((LANES, D), jnp.float32),
        pltpu.VMEM_SHARED((N_SH, D), jnp.float32),
    ],
)


def _agg_body(feat_hbm, row_hbm, col_hbm, zero128_hbm, agg_hbm,
              ridx_v, cidx_v, ridx1, cidx1, rows_v, agg_sh, sem):
    cid = lax.axis_index("c")
    sid = lax.axis_index("s")
    wid = cid * NS + sid

    pltpu.sync_copy(zero128_hbm, agg_sh.at[pl.ds(sid * RPS, RPS)])
    plsc.subcore_barrier()

    pltpu.sync_copy(row_hbm.at[wid], ridx_v)
    pltpu.sync_copy(col_hbm.at[wid], cidx_v)

    def chunk(j, _):
        for i in range(LANES // 16):
            ridx1[pl.ds(i * 16, 16)] = ridx_v[j, pl.ds(i * 16, 16)]
            cidx1[pl.ds(i * 16, 16)] = cidx_v[j, pl.ds(i * 16, 16)]
        pltpu.async_copy(feat_hbm.at[ridx1], rows_v, sem).wait()
        pltpu.sync_copy(rows_v, agg_sh.at[cidx1], add=True)
        return 0

    lax.fori_loop(0, CH, chunk, 0)

    plsc.subcore_barrier()
    pltpu.sync_copy(
        agg_sh.at[pl.ds(sid * RPS, RPS)],
        agg_hbm.at[cid, pl.ds(sid * RPS, RPS)],
    )


_agg_call = pl.kernel(
    _agg_body,
    out_type=jax.ShapeDtypeStruct((NC, N_SH, D), jnp.float32),
    mesh=_MESH,
    scratch_types=[
        pltpu.VMEM((CH, LANES), jnp.int32),
        pltpu.VMEM((CH, LANES), jnp.int32),
        pltpu.VMEM((LANES,), jnp.int32),
        pltpu.VMEM((LANES,), jnp.int32),
        pltpu.VMEM((LANES, D), jnp.float32),
        pltpu.VMEM_SHARED((N_SH, D), jnp.float32),
        pltpu.SemaphoreType.DMA,
    ],
)


def _hs_body(x_ref, w_ref, degp_ref, hs_ref):
    deg = degp_ref[0, :] + degp_ref[1, :] + 1.0
    dis = lax.rsqrt(deg)
    h = lax.dot_general(
        x_ref[:], w_ref[:], (((1,), (1,)), ((), ())),
        preferred_element_type=jnp.float32,
    )
    hs_ref[:, :] = h * dis[:, None]


def _bn_body(aggp_ref, hs_ref, degp_ref, b_ref, gamma_ref, beta_ref, out_ref):
    deg = degp_ref[0, :] + degp_ref[1, :] + 1.0
    dis = lax.rsqrt(deg)
    t = (aggp_ref[0] + aggp_ref[1] + hs_ref[:, :]) * dis[:, None] + b_ref[0, :][None, :]
    t = jnp.maximum(t, 0.0)
    mean = jnp.mean(t, axis=0)
    centered = t - mean[None, :]
    var = jnp.mean(centered * centered, axis=0)
    scale = gamma_ref[0, :] * lax.rsqrt(var + 1e-5)
    out_ref[:, :] = centered * scale[None, :] + beta_ref[0, :][None, :]


def kernel(x, edge_index, edge_attr, W, b, gamma, beta):
    row = edge_index[0].astype(jnp.int32)
    col = edge_index[1].astype(jnp.int32)
    pad = E_PAD - N_EDGES
    rowp = jnp.concatenate([row, jnp.zeros((pad,), jnp.int32)]).reshape(NW, CH, LANES)
    colp = jnp.concatenate([col, jnp.full((pad,), N_TRASH, jnp.int32)]).reshape(
        NW, CH, LANES
    )
    zero128 = jnp.zeros((RPS, D), jnp.float32)

    degc = _deg_call(colp, zero128)
    degp = degc[:, :N_NODES, 0]

    hs = pl.pallas_call(
        _hs_body,
        out_shape=jax.ShapeDtypeStruct((N_NODES, D), jnp.float32),
    )(x, W, degp)

    aggp = _agg_call(hs, rowp, colp, zero128)[:, :N_NODES]

    out = pl.pallas_call(
        _bn_body,
        out_shape=jax.ShapeDtypeStruct((N_NODES, D), jnp.float32),
    )(
        aggp,
        hs,
        degp,
        b.reshape(1, D),
        gamma.reshape(1, D),
        beta.reshape(1, D),
    )
    return out


# speedup vs baseline: 13.0026x; 1.1133x over previous
import jax
import jax.numpy as jnp
from jax import lax
from jax.experimental import pallas as pl
from jax.experimental.pallas import tpu as pltpu
from jax.experimental.pallas import tpu_sc as plsc

N_NODES = 10000
N_EDGES = 320000
D = 128

NC = 2
NS = 16
NW = NC * NS
LANES = 128
CH = 80
CB = 64
NCH = CH * LANES // CB
E_PAD = NW * CH * LANES
N_TRASH = N_NODES
N_SH = 10240
RPS = N_SH // NS

_MESH = plsc.VectorSubcoreMesh(
    core_axis_name="c", subcore_axis_name="s", num_cores=NC, num_subcores=NS
)


def _deg_body(col_hbm, zero128_hbm, deg_hbm, cidx_v, cidx1, ones_v, hist_sh):
    cid = lax.axis_index("c")
    sid = lax.axis_index("s")
    wid = cid * NS + sid

    pltpu.sync_copy(zero128_hbm, hist_sh.at[pl.ds(sid * RPS, RPS)])

    def fill_ones(i, _):
        for k in range(D // 16):
            ones_v[i, pl.ds(k * 16, 16)] = jnp.ones((16,), jnp.float32)
        return 0

    lax.fori_loop(0, LANES, fill_ones, 0)
    plsc.subcore_barrier()

    pltpu.sync_copy(col_hbm.at[wid], cidx_v)

    def chunk(j, _):
        for i in range(LANES // 16):
            cidx1[pl.ds(i * 16, 16)] = cidx_v[j, pl.ds(i * 16, 16)]
        pltpu.sync_copy(ones_v, hist_sh.at[cidx1], add=True)
        return 0

    lax.fori_loop(0, CH, chunk, 0)

    plsc.subcore_barrier()
    pltpu.sync_copy(
        hist_sh.at[pl.ds(sid * RPS, RPS)],
        deg_hbm.at[cid, pl.ds(sid * RPS, RPS)],
    )


_deg_call = pl.kernel(
    _deg_body,
    out_type=jax.ShapeDtypeStruct((NC, N_SH, D), jnp.float32),
    mesh=_MESH,
    scratch_types=[
        pltpu.VMEM((CH, LANES), jnp.int32),
        pltpu.VMEM((LANES,), jnp.int32),
        pltpu.VMEM((LANES, D), jnp.float32),
        pltpu.VMEM_SHARED((N_SH, D), jnp.float32),
    ],
)


def _agg_body(feat_hbm, row_hbm, col_hbm, zero128_hbm, agg_hbm,
              ridx_v, cidx_v, ridx1a, cidx1a, ridx1b, cidx1b,
              rows_a, rows_b, agg_sh, sem_a, sem_b):
    cid = lax.axis_index("c")
    sid = lax.axis_index("s")
    wid = cid * NS + sid

    pltpu.sync_copy(zero128_hbm, agg_sh.at[pl.ds(sid * RPS, RPS)])
    plsc.subcore_barrier()

    pltpu.sync_copy(row_hbm.at[wid], ridx_v)
    pltpu.sync_copy(col_hbm.at[wid], cidx_v)

    def stage(j, ridx1, cidx1):
        jr = j // 2
        jo = (j % 2) * CB
        for i in range(CB // 16):
            ridx1[pl.ds(i * 16, 16)] = ridx_v[jr, pl.ds(jo + i * 16, 16)]
            cidx1[pl.ds(i * 16, 16)] = cidx_v[jr, pl.ds(jo + i * 16, 16)]

    stage(0, ridx1a, cidx1a)
    pltpu.async_copy(feat_hbm.at[ridx1a], rows_a, sem_a)

    def outer(j2, _):
        j = j2 * 2
        stage(j + 1, ridx1b, cidx1b)
        pltpu.async_copy(feat_hbm.at[ridx1b], rows_b, sem_b)
        pltpu.make_async_copy(feat_hbm.at[ridx1a], rows_a, sem_a).wait()
        pltpu.sync_copy(rows_a, agg_sh.at[cidx1a], add=True)

        @pl.when(j + 2 < NCH)
        def _():
            stage(j + 2, ridx1a, cidx1a)
            pltpu.async_copy(feat_hbm.at[ridx1a], rows_a, sem_a)

        pltpu.make_async_copy(feat_hbm.at[ridx1b], rows_b, sem_b).wait()
        pltpu.sync_copy(rows_b, agg_sh.at[cidx1b], add=True)
        return 0

    lax.fori_loop(0, NCH // 2, outer, 0)

    plsc.subcore_barrier()
    pltpu.sync_copy(
        agg_sh.at[pl.ds(sid * RPS, RPS)],
        agg_hbm.at[cid, pl.ds(sid * RPS, RPS)],
    )


_agg_call = pl.kernel(
    _agg_body,
    out_type=jax.ShapeDtypeStruct((NC, N_SH, D), jnp.float32),
    mesh=_MESH,
    scratch_types=[
        pltpu.VMEM((CH, LANES), jnp.int32),
        pltpu.VMEM((CH, LANES), jnp.int32),
        pltpu.VMEM((CB,), jnp.int32),
        pltpu.VMEM((CB,), jnp.int32),
        pltpu.VMEM((CB,), jnp.int32),
        pltpu.VMEM((CB,), jnp.int32),
        pltpu.VMEM((CB, D), jnp.float32),
        pltpu.VMEM((CB, D), jnp.float32),
        pltpu.VMEM_SHARED((N_SH, D), jnp.float32),
        pltpu.SemaphoreType.DMA,
        pltpu.SemaphoreType.DMA,
    ],
)


def _hs_body(x_ref, w_ref, degp_ref, hs_ref):
    deg = degp_ref[0, :] + degp_ref[1, :] + 1.0
    dis = lax.rsqrt(deg)
    h = lax.dot_general(
        x_ref[:], w_ref[:], (((1,), (1,)), ((), ())),
        preferred_element_type=jnp.float32,
    )
    hs_ref[:, :] = h * dis[:, None]


def _bn_body(aggp_ref, hs_ref, degp_ref, b_ref, gamma_ref, beta_ref, out_ref):
    deg = degp_ref[0, :] + degp_ref[1, :] + 1.0
    dis = lax.rsqrt(deg)
    t = (aggp_ref[0] + aggp_ref[1] + hs_ref[:, :]) * dis[:, None] + b_ref[0, :][None, :]
    t = jnp.maximum(t, 0.0)
    mean = jnp.mean(t, axis=0)
    centered = t - mean[None, :]
    var = jnp.mean(centered * centered, axis=0)
    scale = gamma_ref[0, :] * lax.rsqrt(var + 1e-5)
    out_ref[:, :] = centered * scale[None, :] + beta_ref[0, :][None, :]


def kernel(x, edge_index, edge_attr, W, b, gamma, beta):
    row = edge_index[0].astype(jnp.int32)
    col = edge_index[1].astype(jnp.int32)
    pad = E_PAD - N_EDGES
    rowp = jnp.concatenate([row, jnp.zeros((pad,), jnp.int32)]).reshape(NW, CH, LANES)
    colp = jnp.concatenate([col, jnp.full((pad,), N_TRASH, jnp.int32)]).reshape(
        NW, CH, LANES
    )
    zero128 = jnp.zeros((RPS, D), jnp.float32)

    degc = _deg_call(colp, zero128)
    degp = degc[:, :N_NODES, 0]

    hs = pl.pallas_call(
        _hs_body,
        out_shape=jax.ShapeDtypeStruct((N_NODES, D), jnp.float32),
    )(x, W, degp)

    aggp = _agg_call(hs, rowp, colp, zero128)[:, :N_NODES]

    out = pl.pallas_call(
        _bn_body,
        out_shape=jax.ShapeDtypeStruct((N_NODES, D), jnp.float32),
    )(
        aggp,
        hs,
        degp,
        b.reshape(1, D),
        gamma.reshape(1, D),
        beta.reshape(1, D),
    )
    return out
